# baseline (speedup 1.0000x reference)
import functools

import jax
import jax.numpy as jnp
from jax import lax
from jax.experimental import pallas as pl
from jax.experimental.pallas import tpu as pltpu
from jax.experimental.pallas import tpu_sc as plsc

NN = 10000
NP = 10240
DD = 128
GG = 64
EE = 320000
NCORE = 2
NSUB = 16
NWORK = NCORE * NSUB
CHUNK = 128
CPW = 80
EPW = CHUNK * CPW
EPAD = NWORK * EPW
NACC = NP
DUMP = NN
HALF = CPW // 2
BLK = 512
NBLK = NP // BLK

_SC_MESH = plsc.VectorSubcoreMesh(
    core_axis_name="c", subcore_axis_name="s",
    num_cores=NCORE, num_subcores=NSUB)


def _zero_shared(acc_sh, zbuf_v, s, width):
    nch = NACC // CHUNK

    def body(i, carry):
        ch = s + NSUB * i
        pltpu.sync_copy(zbuf_v, acc_sh.at[pl.ds(ch * CHUNK, CHUNK)])
        return carry

    lax.fori_loop(0, nch // NSUB, body, 0)


def _copy_out(acc_sh, out_hbm, c, s):
    rows = NP // NSUB
    pltpu.sync_copy(acc_sh.at[pl.ds(s * rows, rows)],
                    out_hbm.at[c, pl.ds(s * rows, rows)])


def _agg_body(h_hbm, src_hbm, dst_hbm, zeros_hbm, out_hbm,
              src_v, dst_v, rows0_v, rows1_v, acc_sh, sem0, sem1):
    c = lax.axis_index("c")
    s = lax.axis_index("s")
    wid = s * NCORE + c

    pltpu.sync_copy(src_hbm.at[wid, pl.ds(0, HALF)], src_v)
    pltpu.async_copy(h_hbm.at[src_v.at[0]], rows0_v, sem0)
    pltpu.sync_copy(dst_hbm.at[wid, pl.ds(0, HALF)], dst_v)
    pltpu.sync_copy(zeros_hbm, rows1_v)
    _zero_shared(acc_sh, rows1_v, s, DD)
    plsc.subcore_barrier()

    for half in range(2):
        if half > 0:
            pltpu.sync_copy(src_hbm.at[wid, pl.ds(half * HALF, HALF)], src_v)
            pltpu.sync_copy(dst_hbm.at[wid, pl.ds(half * HALF, HALF)], dst_v)
            pltpu.async_copy(h_hbm.at[src_v.at[0]], rows0_v, sem0)

        def edge_pair(gg, carry):
            g = 2 * gg
            pltpu.make_async_copy(h_hbm.at[src_v.at[g]], rows0_v, sem0).wait()
            pltpu.async_copy(h_hbm.at[src_v.at[g + 1]], rows1_v, sem1)
            pltpu.sync_copy(rows0_v, acc_sh.at[dst_v.at[g]], add=True)
            pltpu.make_async_copy(
                h_hbm.at[src_v.at[g + 1]], rows1_v, sem1).wait()

            @pl.when(gg + 1 < HALF // 2)
            def _():
                pltpu.async_copy(h_hbm.at[src_v.at[g + 2]], rows0_v, sem0)

            pltpu.sync_copy(rows1_v, acc_sh.at[dst_v.at[g + 1]], add=True)
            return carry

        lax.fori_loop(0, HALF // 2, edge_pair, 0)
    plsc.subcore_barrier()
    _copy_out(acc_sh, out_hbm, c, s)


_agg_call = pl.kernel(
    _agg_body,
    out_type=jax.ShapeDtypeStruct((NCORE, NP, DD), jnp.float32),
    mesh=_SC_MESH,
    scratch_types=[
        pltpu.VMEM((HALF, CHUNK), jnp.int32),
        pltpu.VMEM((HALF, CHUNK), jnp.int32),
        pltpu.VMEM((CHUNK, DD), jnp.float32),
        pltpu.VMEM((CHUNK, DD), jnp.float32),
        pltpu.VMEM_SHARED((NACC, DD), jnp.float32),
        pltpu.SemaphoreType.DMA,
        pltpu.SemaphoreType.DMA,
    ],
)


def _deg_body(dst_hbm, zeros_hbm, ones_hbm, out_hbm,
              dst_v, buf_v, ones_v, acc_sh):
    c = lax.axis_index("c")
    s = lax.axis_index("s")
    wid = s * NCORE + c
    pltpu.sync_copy(dst_hbm.at[wid], dst_v)
    pltpu.sync_copy(ones_hbm, ones_v)
    pltpu.sync_copy(zeros_hbm, buf_v)
    _zero_shared(acc_sh, buf_v, s, DD)
    plsc.subcore_barrier()

    def edge_chunk(g, carry):
        pltpu.sync_copy(ones_v, acc_sh.at[dst_v.at[g]], add=True)
        return carry

    lax.fori_loop(0, CPW, edge_chunk, 0)
    plsc.subcore_barrier()
    _copy_out(acc_sh, out_hbm, c, s)


_deg_call = pl.kernel(
    _deg_body,
    out_type=jax.ShapeDtypeStruct((NCORE, NP, DD), jnp.float32),
    mesh=_SC_MESH,
    scratch_types=[
        pltpu.VMEM((CPW, CHUNK), jnp.int32),
        pltpu.VMEM((CHUNK, DD), jnp.float32),
        pltpu.VMEM((CHUNK, DD), jnp.float32),
        pltpu.VMEM_SHARED((NACC, DD), jnp.float32),
    ],
)


def _dinv(d0_ref, d1_ref):
    deg = d0_ref[:, 0:1] + d1_ref[:, 0:1] + 1.0
    return lax.rsqrt(deg)


def _mmu_body(x_ref, w_ref, u_ref):
    u_ref[...] = jnp.dot(x_ref[...], w_ref[...],
                         preferred_element_type=jnp.float32)


def _scale_body(u_ref, d0_ref, d1_ref, h_ref):
    h_ref[...] = u_ref[...] * _dinv(d0_ref, d1_ref)


def _mid1_body(h_ref, a0_ref, a1_ref, d0_ref, d1_ref, b_ref, w_ref,
               hn_ref, xc_ref):
    dinv = _dinv(d0_ref, d1_ref)
    xc = jnp.maximum(
        (h_ref[...] + a0_ref[...] + a1_ref[...]) * dinv + b_ref[...], 0.0)
    xc_ref[...] = xc
    hn_ref[...] = jnp.dot(xc, w_ref[...],
                          preferred_element_type=jnp.float32) * dinv


def _mid2_body(h_ref, a0_ref, a1_ref, d0_ref, d1_ref, b_ref, xp_ref, w_ref,
               hn_ref):
    dinv = _dinv(d0_ref, d1_ref)
    xc = jnp.maximum(
        (h_ref[...] + a0_ref[...] + a1_ref[...]) * dinv + b_ref[...], 0.0)
    xin = xc + xp_ref[...]
    hn_ref[...] = jnp.dot(xin, w_ref[...],
                          preferred_element_type=jnp.float32) * dinv


def _fin_body(h_ref, a0_ref, a1_ref, d0_ref, d1_ref, b_ref, bt_ref,
              wl_ref, bl_ref, y_ref, sums, cnts):
    i = pl.program_id(0)

    @pl.when(i == 0)
    def _():
        sums[...] = jnp.zeros_like(sums)
        cnts[...] = jnp.zeros_like(cnts)

    dinv = _dinv(d0_ref, d1_ref)
    x3 = jnp.maximum(
        (h_ref[...] + a0_ref[...] + a1_ref[...]) * dinv + b_ref[...], 0.0)
    bt = bt_ref[0, 0, :]
    seg = lax.broadcasted_iota(jnp.int32, (GG, BLK), 0)
    oh = (bt[None, :] == seg).astype(jnp.float32)
    sums[...] += jnp.dot(oh, x3, preferred_element_type=jnp.float32)
    cnts[...] += jnp.broadcast_to(jnp.sum(oh, axis=1, keepdims=True),
                                  (GG, DD))

    @pl.when(i == pl.num_programs(0) - 1)
    def _():
        pooled = sums[...] / jnp.maximum(cnts[...], 1.0)
        y_ref[...] = jnp.dot(pooled, wl_ref[...],
                             preferred_element_type=jnp.float32) + bl_ref[...]


_row = lambda i: (i, 0)
_fix = lambda i: (0, 0)

_mmu = pl.pallas_call(
    _mmu_body,
    grid=(NBLK,),
    in_specs=[
        pl.BlockSpec((BLK, DD), _row),
        pl.BlockSpec((DD, DD), _fix),
    ],
    out_specs=pl.BlockSpec((BLK, DD), _row),
    out_shape=jax.ShapeDtypeStruct((NP, DD), jnp.float32),
)

_scale = pl.pallas_call(
    _scale_body,
    grid=(NBLK,),
    in_specs=[
        pl.BlockSpec((BLK, DD), _row),
        pl.BlockSpec((BLK, DD), _row),
        pl.BlockSpec((BLK, DD), _row),
    ],
    out_specs=pl.BlockSpec((BLK, DD), _row),
    out_shape=jax.ShapeDtypeStruct((NP, DD), jnp.float32),
)

_mid1 = pl.pallas_call(
    _mid1_body,
    grid=(NBLK,),
    in_specs=[
        pl.BlockSpec((BLK, DD), _row),
        pl.BlockSpec((BLK, DD), _row),
        pl.BlockSpec((BLK, DD), _row),
        pl.BlockSpec((BLK, DD), _row),
        pl.BlockSpec((BLK, DD), _row),
        pl.BlockSpec((1, DD), _fix),
        pl.BlockSpec((DD, DD), _fix),
    ],
    out_specs=[pl.BlockSpec((BLK, DD), _row), pl.BlockSpec((BLK, DD), _row)],
    out_shape=[jax.ShapeDtypeStruct((NP, DD), jnp.float32),
               jax.ShapeDtypeStruct((NP, DD), jnp.float32)],
)

_mid2 = pl.pallas_call(
    _mid2_body,
    grid=(NBLK,),
    in_specs=[
        pl.BlockSpec((BLK, DD), _row),
        pl.BlockSpec((BLK, DD), _row),
        pl.BlockSpec((BLK, DD), _row),
        pl.BlockSpec((BLK, DD), _row),
        pl.BlockSpec((BLK, DD), _row),
        pl.BlockSpec((1, DD), _fix),
        pl.BlockSpec((BLK, DD), _row),
        pl.BlockSpec((DD, DD), _fix),
    ],
    out_specs=pl.BlockSpec((BLK, DD), _row),
    out_shape=jax.ShapeDtypeStruct((NP, DD), jnp.float32),
)

_fin = pl.pallas_call(
    _fin_body,
    grid=(NBLK,),
    in_specs=[
        pl.BlockSpec((BLK, DD), _row),
        pl.BlockSpec((BLK, DD), _row),
        pl.BlockSpec((BLK, DD), _row),
        pl.BlockSpec((BLK, DD), _row),
        pl.BlockSpec((BLK, DD), _row),
        pl.BlockSpec((1, DD), _fix),
        pl.BlockSpec((1, 1, BLK), lambda i: (i, 0, 0)),
        pl.BlockSpec((DD, 16), _fix),
        pl.BlockSpec((1, 16), _fix),
    ],
    out_specs=pl.BlockSpec((GG, 16), _fix),
    out_shape=jax.ShapeDtypeStruct((GG, 16), jnp.float32),
    scratch_shapes=[pltpu.VMEM((GG, DD), jnp.float32),
                    pltpu.VMEM((GG, DD), jnp.float32)],
)


def kernel(x, edge_index, batch, W0, b0, W1, b1, W2, b2, Wl, bl):
    src = edge_index[0]
    dst = edge_index[1]
    pad_e = EPAD - EE
    pad_src = (jnp.arange(pad_e, dtype=jnp.int32) * 37) % NN
    srcp = jnp.concatenate([src, pad_src]).reshape(NWORK, CPW, CHUNK)
    dump_rows = DUMP + (jnp.arange(pad_e, dtype=jnp.int32) % CHUNK)
    dstp = jnp.concatenate([dst, dump_rows]).reshape(NWORK, CPW, CHUNK)
    xp = jnp.pad(x, ((0, NP - NN), (0, 0)))
    batp = jnp.pad(batch, (0, NP - NN),
                   constant_values=GG).reshape(NBLK, 1, BLK)
    zeros128 = jnp.zeros((CHUNK, DD), jnp.float32)
    ones128 = jnp.ones((CHUNK, DD), jnp.float32)

    u = _mmu(xp, W0)
    degp = _deg_call(dstp, zeros128, ones128)
    d0, d1 = degp[0], degp[1]
    h0 = _scale(u, d0, d1)
    a0 = _agg_call(h0, srcp, dstp, zeros128)
    h1, x1 = _mid1(h0, a0[0], a0[1], d0, d1, b0.reshape(1, DD), W1)
    a1 = _agg_call(h1, srcp, dstp, zeros128)
    h2 = _mid2(h1, a1[0], a1[1], d0, d1, b1.reshape(1, DD), x1, W2)
    a2 = _agg_call(h2, srcp, dstp, zeros128)
    y = _fin(h2, a2[0], a2[1], d0, d1, b2.reshape(1, DD), batp,
             Wl, bl.reshape(1, 16))
    return y


# speedup vs baseline: 1.0014x; 1.0014x over previous
import jax
import jax.numpy as jnp
from jax import lax
from jax.experimental import pallas as pl
from jax.experimental.pallas import tpu as pltpu
from jax.experimental.pallas import tpu_sc as plsc

NN = 10000
NP = 10240
DD = 128
GG = 64
EE = 320000
NCORE = 2
NSUB = 16
NWORK = NCORE * NSUB
CHUNK = 128
CPW = 80
EPW = CHUNK * CPW
EPAD = NWORK * EPW
NACC = NP
DUMP = NN
HALF = CPW // 2
BLK = 512
NBLK = NP // BLK

_SC_MESH = plsc.VectorSubcoreMesh(
    core_axis_name="c", subcore_axis_name="s",
    num_cores=NCORE, num_subcores=NSUB)


def _zero_shared(acc_sh, zbuf_v, s, width):
    nch = NACC // CHUNK

    def body(i, carry):
        ch = s + NSUB * i
        pltpu.sync_copy(zbuf_v, acc_sh.at[pl.ds(ch * CHUNK, CHUNK)])
        return carry

    lax.fori_loop(0, nch // NSUB, body, 0)


def _copy_out(acc_sh, out_hbm, c, s):
    rows = NP // NSUB
    pltpu.sync_copy(acc_sh.at[pl.ds(s * rows, rows)],
                    out_hbm.at[c, pl.ds(s * rows, rows)])


def _agg_body(h_hbm, src_hbm, dst_hbm, zeros_hbm, out_hbm,
              src_v, dst_v, rows0_v, rows1_v, acc_sh, sem0, sem1):
    c = lax.axis_index("c")
    s = lax.axis_index("s")
    wid = s * NCORE + c

    pltpu.sync_copy(src_hbm.at[wid, pl.ds(0, HALF)], src_v)
    pltpu.async_copy(h_hbm.at[src_v.at[0]], rows0_v, sem0)
    pltpu.sync_copy(dst_hbm.at[wid, pl.ds(0, HALF)], dst_v)
    pltpu.sync_copy(zeros_hbm, rows1_v)
    _zero_shared(acc_sh, rows1_v, s, DD)
    plsc.subcore_barrier()

    for half in range(2):
        if half > 0:
            pltpu.sync_copy(src_hbm.at[wid, pl.ds(half * HALF, HALF)], src_v)
            pltpu.sync_copy(dst_hbm.at[wid, pl.ds(half * HALF, HALF)], dst_v)
            pltpu.async_copy(h_hbm.at[src_v.at[0]], rows0_v, sem0)

        def edge_pair(gg, carry):
            g = 2 * gg
            pltpu.make_async_copy(h_hbm.at[src_v.at[g]], rows0_v, sem0).wait()
            pltpu.async_copy(h_hbm.at[src_v.at[g + 1]], rows1_v, sem1)
            pltpu.sync_copy(rows0_v, acc_sh.at[dst_v.at[g]], add=True)
            pltpu.make_async_copy(
                h_hbm.at[src_v.at[g + 1]], rows1_v, sem1).wait()

            @pl.when(gg + 1 < HALF // 2)
            def _():
                pltpu.async_copy(h_hbm.at[src_v.at[g + 2]], rows0_v, sem0)

            pltpu.sync_copy(rows1_v, acc_sh.at[dst_v.at[g + 1]], add=True)
            return carry

        lax.fori_loop(0, HALF // 2, edge_pair, 0)
    plsc.subcore_barrier()
    _copy_out(acc_sh, out_hbm, c, s)


_agg_call = pl.kernel(
    _agg_body,
    out_type=jax.ShapeDtypeStruct((NCORE, NP, DD), jnp.float32),
    mesh=_SC_MESH,
    scratch_types=[
        pltpu.VMEM((HALF, CHUNK), jnp.int32),
        pltpu.VMEM((HALF, CHUNK), jnp.int32),
        pltpu.VMEM((CHUNK, DD), jnp.float32),
        pltpu.VMEM((CHUNK, DD), jnp.float32),
        pltpu.VMEM_SHARED((NACC, DD), jnp.float32),
        pltpu.SemaphoreType.DMA,
        pltpu.SemaphoreType.DMA,
    ],
)


def _deg_body(dst_hbm, zeros_hbm, ones_hbm, out_hbm,
              dst_v, buf_v, ones_v, acc_sh):
    c = lax.axis_index("c")
    s = lax.axis_index("s")
    wid = s * NCORE + c
    pltpu.sync_copy(dst_hbm.at[wid], dst_v)
    pltpu.sync_copy(ones_hbm, ones_v)
    pltpu.sync_copy(zeros_hbm, buf_v)
    _zero_shared(acc_sh, buf_v, s, DD)
    plsc.subcore_barrier()

    def edge_chunk(g, carry):
        pltpu.sync_copy(ones_v, acc_sh.at[dst_v.at[g]], add=True)
        return carry

    lax.fori_loop(0, CPW, edge_chunk, 0)
    plsc.subcore_barrier()
    _copy_out(acc_sh, out_hbm, c, s)


_deg_call = pl.kernel(
    _deg_body,
    out_type=jax.ShapeDtypeStruct((NCORE, NP, DD), jnp.float32),
    mesh=_SC_MESH,
    scratch_types=[
        pltpu.VMEM((CPW, CHUNK), jnp.int32),
        pltpu.VMEM((CHUNK, DD), jnp.float32),
        pltpu.VMEM((CHUNK, DD), jnp.float32),
        pltpu.VMEM_SHARED((NACC, DD), jnp.float32),
    ],
)


def _dinv(d0_ref, d1_ref):
    deg = d0_ref[:, 0:1] + d1_ref[:, 0:1] + 1.0
    return lax.rsqrt(deg)


def _mmu_body(x_ref, w_ref, u_ref):
    u_ref[...] = jnp.dot(x_ref[...], w_ref[...],
                         preferred_element_type=jnp.float32)


def _scale_body(u_ref, d0_ref, d1_ref, h_ref):
    h_ref[...] = u_ref[...] * _dinv(d0_ref, d1_ref)


def _mid1_body(h_ref, a0_ref, a1_ref, d0_ref, d1_ref, b_ref, w_ref,
               hn_ref, xc_ref):
    dinv = _dinv(d0_ref, d1_ref)
    xc = jnp.maximum(
        (h_ref[...] + a0_ref[...] + a1_ref[...]) * dinv + b_ref[...], 0.0)
    xc_ref[...] = xc
    hn_ref[...] = jnp.dot(xc, w_ref[...],
                          preferred_element_type=jnp.float32) * dinv


def _mid2_body(h_ref, a0_ref, a1_ref, d0_ref, d1_ref, b_ref, xp_ref, w_ref,
               hn_ref):
    dinv = _dinv(d0_ref, d1_ref)
    xc = jnp.maximum(
        (h_ref[...] + a0_ref[...] + a1_ref[...]) * dinv + b_ref[...], 0.0)
    xin = xc + xp_ref[...]
    hn_ref[...] = jnp.dot(xin, w_ref[...],
                          preferred_element_type=jnp.float32) * dinv


def _fin_body(h_ref, a0_ref, a1_ref, d0_ref, d1_ref, b_ref, bt_ref,
              wl_ref, bl_ref, y_ref, sums, cnts):
    i = pl.program_id(0)

    @pl.when(i == 0)
    def _():
        sums[...] = jnp.zeros_like(sums)
        cnts[...] = jnp.zeros_like(cnts)

    dinv = _dinv(d0_ref, d1_ref)
    x3 = jnp.maximum(
        (h_ref[...] + a0_ref[...] + a1_ref[...]) * dinv + b_ref[...], 0.0)
    bt = bt_ref[0, 0, :]
    seg = lax.broadcasted_iota(jnp.int32, (GG, BLK), 0)
    oh = (bt[None, :] == seg).astype(jnp.float32)
    sums[...] += jnp.dot(oh, x3, preferred_element_type=jnp.float32)
    cnts[...] += jnp.broadcast_to(jnp.sum(oh, axis=1, keepdims=True),
                                  (GG, DD))

    @pl.when(i == pl.num_programs(0) - 1)
    def _():
        pooled = sums[...] / jnp.maximum(cnts[...], 1.0)
        y_ref[...] = jnp.dot(pooled, wl_ref[...],
                             preferred_element_type=jnp.float32) + bl_ref[...]


_row = lambda i: (i, 0)
_fix = lambda i: (0, 0)

_mmu = pl.pallas_call(
    _mmu_body,
    grid=(NBLK,),
    in_specs=[
        pl.BlockSpec((BLK, DD), _row),
        pl.BlockSpec((DD, DD), _fix),
    ],
    out_specs=pl.BlockSpec((BLK, DD), _row),
    out_shape=jax.ShapeDtypeStruct((NP, DD), jnp.float32),
)

_scale = pl.pallas_call(
    _scale_body,
    grid=(NBLK,),
    in_specs=[
        pl.BlockSpec((BLK, DD), _row),
        pl.BlockSpec((BLK, DD), _row),
        pl.BlockSpec((BLK, DD), _row),
    ],
    out_specs=pl.BlockSpec((BLK, DD), _row),
    out_shape=jax.ShapeDtypeStruct((NP, DD), jnp.float32),
)

_mid1 = pl.pallas_call(
    _mid1_body,
    grid=(NBLK,),
    in_specs=[
        pl.BlockSpec((BLK, DD), _row),
        pl.BlockSpec((BLK, DD), _row),
        pl.BlockSpec((BLK, DD), _row),
        pl.BlockSpec((BLK, DD), _row),
        pl.BlockSpec((BLK, DD), _row),
        pl.BlockSpec((1, DD), _fix),
        pl.BlockSpec((DD, DD), _fix),
    ],
    out_specs=[pl.BlockSpec((BLK, DD), _row), pl.BlockSpec((BLK, DD), _row)],
    out_shape=[jax.ShapeDtypeStruct((NP, DD), jnp.float32),
               jax.ShapeDtypeStruct((NP, DD), jnp.float32)],
)

_mid2 = pl.pallas_call(
    _mid2_body,
    grid=(NBLK,),
    in_specs=[
        pl.BlockSpec((BLK, DD), _row),
        pl.BlockSpec((BLK, DD), _row),
        pl.BlockSpec((BLK, DD), _row),
        pl.BlockSpec((BLK, DD), _row),
        pl.BlockSpec((BLK, DD), _row),
        pl.BlockSpec((1, DD), _fix),
        pl.BlockSpec((BLK, DD), _row),
        pl.BlockSpec((DD, DD), _fix),
    ],
    out_specs=pl.BlockSpec((BLK, DD), _row),
    out_shape=jax.ShapeDtypeStruct((NP, DD), jnp.float32),
)

_fin = pl.pallas_call(
    _fin_body,
    grid=(NBLK,),
    in_specs=[
        pl.BlockSpec((BLK, DD), _row),
        pl.BlockSpec((BLK, DD), _row),
        pl.BlockSpec((BLK, DD), _row),
        pl.BlockSpec((BLK, DD), _row),
        pl.BlockSpec((BLK, DD), _row),
        pl.BlockSpec((1, DD), _fix),
        pl.BlockSpec((1, 1, BLK), lambda i: (i, 0, 0)),
        pl.BlockSpec((DD, 16), _fix),
        pl.BlockSpec((1, 16), _fix),
    ],
    out_specs=pl.BlockSpec((GG, 16), _fix),
    out_shape=jax.ShapeDtypeStruct((GG, 16), jnp.float32),
    scratch_shapes=[pltpu.VMEM((GG, DD), jnp.float32),
                    pltpu.VMEM((GG, DD), jnp.float32)],
)


def kernel(x, edge_index, batch, W0, b0, W1, b1, W2, b2, Wl, bl):
    src = edge_index[0]
    dst = edge_index[1]
    pad_e = EPAD - EE
    pad_src = (jnp.arange(pad_e, dtype=jnp.int32) * 37) % NN
    srcp = jnp.concatenate([src, pad_src]).reshape(NWORK, CPW, CHUNK)
    dump_rows = DUMP + (jnp.arange(pad_e, dtype=jnp.int32) % CHUNK)
    dstp = jnp.concatenate([dst, dump_rows]).reshape(NWORK, CPW, CHUNK)
    xp = jnp.pad(x, ((0, NP - NN), (0, 0)))
    batp = jnp.pad(batch, (0, NP - NN),
                   constant_values=GG).reshape(NBLK, 1, BLK)
    zeros128 = jnp.zeros((CHUNK, DD), jnp.float32)
    ones128 = jnp.ones((CHUNK, DD), jnp.float32)

    u = _mmu(xp, W0)
    degp = _deg_call(dstp, zeros128, ones128)
    d0, d1 = degp[0], degp[1]
    h0 = _scale(u, d0, d1)
    a0 = _agg_call(h0, srcp, dstp, zeros128)
    h1, x1 = _mid1(h0, a0[0], a0[1], d0, d1, b0.reshape(1, DD), W1)
    a1 = _agg_call(h1, srcp, dstp, zeros128)
    h2 = _mid2(h1, a1[0], a1[1], d0, d1, b1.reshape(1, DD), x1, W2)
    a2 = _agg_call(h2, srcp, dstp, zeros128)
    y = _fin(h2, a2[0], a2[1], d0, d1, b2.reshape(1, DD), batp,
             Wl, bl.reshape(1, 16))
    return y
